# Initial kernel scaffold; baseline (speedup 1.0000x reference)
import jax
import jax.numpy as jnp
from jax.experimental import pallas as pl


def kernel(x, tok_table, pos_table):
    raise NotImplementedError("write your pallas kernel here")


# speedup vs baseline: 2.4025x; 2.4025x over previous
import functools

import jax
import jax.numpy as jnp
from jax import lax
from jax.experimental import pallas as pl
from jax.experimental.pallas import tpu as pltpu
from jax.experimental.pallas import tpu_sc as plsc

B, L, D = 1024, 200, 64
NC, NS = 2, 16
NW = NC * NS
ROWS = B * L
CHUNK = 128
RPW = ROWS // NW
NCH = RPW // CHUNK
NBUF = 3
LANES = 16
KREG = D // LANES


def _body(tok_hbm, idx_hbm, pos_hbm, out_hbm,
          idx_v, pos_v, b0, b1, b2, g0, g1, g2, w0, w1, w2):
    wid = lax.axis_index("s") * NC + lax.axis_index("c")
    base_row = wid * RPW

    pltpu.sync_copy(idx_hbm.at[wid], idx_v)
    pltpu.sync_copy(pos_hbm, pos_v)

    bufs = (b0, b1, b2)
    gsem = (g0, g1, g2)
    wsem = (w0, w1, w2)

    def fire_gather(c):
        nb = c % NBUF
        return pltpu.async_copy(tok_hbm.at[idx_v.at[c]], bufs[nb], gsem[nb])

    gh = {0: fire_gather(0), 1: fire_gather(1)}
    wh = {}
    for c in range(NCH):
        nb = c % NBUF
        gh.pop(c).wait()
        buf = bufs[nb]

        def r_body(r, carry, _c=c, _buf=buf):
            labs = lax.rem(_c * CHUNK + r, L)
            for k in range(KREG):
                sl = pl.ds(k * LANES, LANES)
                _buf[r, sl] = _buf[r, sl] + pos_v[labs, sl]
            return carry

        lax.fori_loop(0, CHUNK, r_body, 0, unroll=2)

        wh[c] = pltpu.async_copy(
            buf, out_hbm.at[pl.ds(base_row + c * CHUNK, CHUNK)], wsem[nb])
        cn = c + 2
        if cn < NCH:
            prev = cn - NBUF
            if prev in wh:
                wh.pop(prev).wait()
            gh[cn] = fire_gather(cn)
    for c in sorted(wh):
        wh.pop(c).wait()


@functools.partial(jax.jit, static_argnames=())
def _embed(tok_table, idx2d, pos):
    run = pl.kernel(
        _body,
        mesh=plsc.VectorSubcoreMesh(core_axis_name="c", subcore_axis_name="s"),
        compiler_params=pltpu.CompilerParams(use_tc_tiling_on_sc=False),
        out_type=jax.ShapeDtypeStruct((ROWS, D), jnp.float32),
        scratch_types=[
            pltpu.VMEM((NCH, CHUNK), jnp.int32),
            pltpu.VMEM((L, D), jnp.float32),
            pltpu.VMEM((CHUNK, D), jnp.float32),
            pltpu.VMEM((CHUNK, D), jnp.float32),
            pltpu.VMEM((CHUNK, D), jnp.float32),
            pltpu.SemaphoreType.DMA,
            pltpu.SemaphoreType.DMA,
            pltpu.SemaphoreType.DMA,
            pltpu.SemaphoreType.DMA,
            pltpu.SemaphoreType.DMA,
            pltpu.SemaphoreType.DMA,
        ],
    )
    return run(tok_table, idx2d, pos)


def kernel(x, tok_table, pos_table):
    idx2d = x.astype(jnp.int32).reshape(NW, NCH, CHUNK)
    pos = pos_table[:L]
    out = _embed(tok_table, idx2d, pos)
    return out.reshape(B, L, D)


# speedup vs baseline: 8.1188x; 3.3793x over previous
import functools

import jax
import jax.numpy as jnp
from jax import lax
from jax.experimental import pallas as pl
from jax.experimental.pallas import tpu as pltpu
from jax.experimental.pallas import tpu_sc as plsc

B, L, D = 1024, 200, 64
NC, NS = 2, 16
NW = NC * NS
BT = 128
NBT = B // BT
UNITS = L * NBT
UPW = UNITS // NW
LANES = 16
BG = BT // LANES
DT = D // 8


def _body(tok_hbm, idx_hbm, pos_hbm, out_hbm,
          idx_v, pos_v, g0, g1, o0, o1, gs0, gs1, ws0, ws1):
    wid = lax.axis_index("s") * NC + lax.axis_index("c")
    base_u = wid * UPW

    pltpu.sync_copy(idx_hbm.at[wid], idx_v.at[pl.ds(0, UPW)])
    pltpu.sync_copy(pos_hbm, pos_v)

    gbuf = (g0, g1)
    obuf = (o0, o1)
    gsem = (gs0, gs1)
    wsem = (ws0, ws1)

    for k in range(BT // LANES):
        idx_v[UPW, pl.ds(k * LANES, LANES)] = jnp.zeros((LANES,), jnp.int32)
        idx_v[UPW + 1, pl.ds(k * LANES, LANES)] = jnp.zeros((LANES,), jnp.int32)

    iota = lax.iota(jnp.int32, LANES)
    rowsel = [iota + (bg * LANES) for bg in range(BG)]

    def fire_gather(i, p):
        pltpu.async_copy(tok_hbm.at[idx_v.at[i]], gbuf[p], gsem[p])

    def wait_gather(p):
        pltpu.make_async_copy(tok_hbm.at[idx_v.at[0]], gbuf[p], gsem[p]).wait()

    def fire_writes(p, l, bt):
        for dt in range(DT):
            pltpu.async_copy(obuf[p].at[pl.ds(dt * 8, 8)],
                             out_hbm.at[l, dt, bt], wsem[p])

    def drain_writes(p):
        for dt in range(DT):
            pltpu.make_async_copy(obuf[p].at[pl.ds(dt * 8, 8)],
                                  out_hbm.at[0, dt, 0], wsem[p]).wait()

    def compute(p, u):
        l = u // NBT
        bt = lax.rem(u, NBT)
        g = gbuf[p]
        o = obuf[p]

        def d_body(d, carry):
            dsel = jnp.full((LANES,), d, jnp.int32)
            lsel = jnp.full((LANES,), l, jnp.int32)
            padd = plsc.load_gather(pos_v, [lsel, dsel])
            for bg in range(BG):
                v = plsc.load_gather(g, [rowsel[bg], dsel])
                o[d, pl.ds(bg * LANES, LANES)] = v + padd
            return carry

        lax.fori_loop(0, D, d_body, 0, unroll=2)
        fire_writes(p, l, bt)

    fire_gather(0, 0)
    fire_gather(1, 1)
    wait_gather(0)
    compute(0, base_u)
    fire_gather(2, 0)
    wait_gather(1)
    compute(1, base_u + 1)

    def j_body(j, carry):
        a = 2 * j
        fire_gather(a + 1, 1)
        wait_gather(0)
        drain_writes(0)
        compute(0, base_u + a)
        fire_gather(a + 2, 0)
        wait_gather(1)
        drain_writes(1)
        compute(1, base_u + a + 1)
        return carry

    lax.fori_loop(1, UPW // 2, j_body, 0)

    wait_gather(0)
    drain_writes(0)
    drain_writes(1)


@jax.jit
def _embed(tok_table, idx3, pos):
    run = pl.kernel(
        _body,
        mesh=plsc.VectorSubcoreMesh(core_axis_name="c", subcore_axis_name="s"),
        compiler_params=pltpu.CompilerParams(
            use_tc_tiling_on_sc=False, needs_layout_passes=False),
        out_type=jax.ShapeDtypeStruct((L, DT, NBT, 8, BT), jnp.float32),
        scratch_types=[
            pltpu.VMEM((UPW + 2, BT), jnp.int32),
            pltpu.VMEM((L, D), jnp.float32),
            pltpu.VMEM((BT, D), jnp.float32),
            pltpu.VMEM((BT, D), jnp.float32),
            pltpu.VMEM((D, BT), jnp.float32),
            pltpu.VMEM((D, BT), jnp.float32),
            pltpu.SemaphoreType.DMA,
            pltpu.SemaphoreType.DMA,
            pltpu.SemaphoreType.DMA,
            pltpu.SemaphoreType.DMA,
        ],
    )
    return run(tok_table, idx3, pos)


def kernel(x, tok_table, pos_table):
    idx3 = jnp.swapaxes(x.astype(jnp.int32), 0, 1).reshape(NW, UPW, BT)
    pos = pos_table[:L]
    out5 = _embed(tok_table, idx3, pos)
    return out5.transpose(2, 4, 0, 1, 3).reshape(B, L, D)
